# Initial kernel scaffold; baseline (speedup 1.0000x reference)
import jax
import jax.numpy as jnp
from jax.experimental import pallas as pl


def kernel(x, edge_index, global_features, batch, params):
    raise NotImplementedError("write your pallas kernel here")


# speedup vs baseline: 15.1700x; 15.1700x over previous
import functools

import jax
import jax.numpy as jnp
from jax import lax
from jax.experimental import pallas as pl
from jax.experimental.pallas import tpu as pltpu
from jax.experimental.pallas import tpu_sc as plsc

NC = 2
NS = 16
WD = 50
WA = 100


def _sc_degree(d2d, zeros1, npad, nwin):
    mesh = plsc.VectorSubcoreMesh(core_axis_name="c", subcore_axis_name="s")
    chunk = npad // NS

    @functools.partial(
        pl.kernel,
        out_type=jax.ShapeDtypeStruct((NC, npad), jnp.float32),
        mesh=mesh,
        scratch_types=[
            pltpu.VMEM((nwin, WD), jnp.int32),
            pltpu.VMEM((WD,), jnp.float32),
            pltpu.VMEM_SHARED((npad,), jnp.float32),
        ],
    )
    def k(d_hbm, z_hbm, out_hbm, didx, ones, deg_sh):
        c = lax.axis_index("c")
        s = lax.axis_index("s")
        wid = c * NS + s

        pltpu.sync_copy(d_hbm.at[pl.ds(wid * nwin, nwin)], didx)
        for i in range(WD // 16):
            ones[pl.ds(i * 16, 16)] = jnp.ones((16,), jnp.float32)
        ones[pl.ds(WD - 16, 16)] = jnp.ones((16,), jnp.float32)
        pltpu.sync_copy(z_hbm.at[pl.ds(s * chunk, chunk)],
                        deg_sh.at[pl.ds(s * chunk, chunk)])
        plsc.subcore_barrier()

        def body(w, carry):
            pltpu.sync_copy(ones, deg_sh.at[didx.at[w]], add=True)
            return carry

        lax.fori_loop(0, nwin, body, 0)
        plsc.subcore_barrier()
        pltpu.sync_copy(deg_sh.at[pl.ds(s * chunk, chunk)],
                        out_hbm.at[c, pl.ds(s * chunk, chunk)])

    return k(d2d, zeros1)


def _sc_aggregate(y, s2d, d2c, zeros2, n, f, nwin):
    mesh = plsc.VectorSubcoreMesh(core_axis_name="c", subcore_axis_name="s")
    half = n // 2
    arows = half + NS
    chunk = (half // NS) // 8 * 8
    last = half - chunk * (NS - 1)
    zchunk = (arows // NS) // 8 * 8
    zlast = arows - zchunk * (NS - 1)

    @functools.partial(
        pl.kernel,
        out_type=jax.ShapeDtypeStruct((NC, half, f), jnp.float32),
        mesh=mesh,
        scratch_types=[
            pltpu.VMEM((nwin, WA), jnp.int32),
            pltpu.VMEM((nwin, WA), jnp.int32),
            pltpu.VMEM((WA, f), jnp.float32),
            pltpu.VMEM((WA, f), jnp.float32),
            pltpu.VMEM_SHARED((arows, f), jnp.float32),
            pltpu.SemaphoreType.DMA,
            pltpu.SemaphoreType.DMA,
        ],
    )
    def k(y_hbm, s_hbm, d_hbm, z_hbm, out_hbm,
          sidx, didx, bufa, bufb, agg_sh, sema, semb):
        c = lax.axis_index("c")
        s = lax.axis_index("s")

        pltpu.sync_copy(s_hbm.at[pl.ds(s * nwin, nwin)], sidx)
        pltpu.sync_copy(d_hbm.at[c, pl.ds(s * nwin, nwin)], didx)

        @pl.when(s < NS - 1)
        def _():
            pltpu.sync_copy(z_hbm.at[pl.ds(s * zchunk, zchunk)],
                            agg_sh.at[pl.ds(s * zchunk, zchunk)])

        @pl.when(s == NS - 1)
        def _():
            pltpu.sync_copy(z_hbm.at[pl.ds((NS - 1) * zchunk, zlast)],
                            agg_sh.at[pl.ds((NS - 1) * zchunk, zlast)])

        plsc.subcore_barrier()

        pltpu.async_copy(y_hbm.at[sidx.at[0]], bufa, sema)
        pltpu.async_copy(y_hbm.at[sidx.at[1]], bufb, semb)

        def body(g, carry):
            w = 2 * g
            pltpu.make_async_copy(y_hbm.at[sidx.at[w]], bufa, sema).wait()
            pltpu.sync_copy(bufa, agg_sh.at[didx.at[w]], add=True)

            @pl.when(w + 2 < nwin)
            def _():
                pltpu.async_copy(y_hbm.at[sidx.at[w + 2]], bufa, sema)

            pltpu.make_async_copy(y_hbm.at[sidx.at[w + 1]], bufb, semb).wait()
            pltpu.sync_copy(bufb, agg_sh.at[didx.at[w + 1]], add=True)

            @pl.when(w + 3 < nwin)
            def _():
                pltpu.async_copy(y_hbm.at[sidx.at[w + 3]], bufb, semb)

            return carry

        lax.fori_loop(0, nwin // 2, body, 0)
        if nwin % 2:
            w = nwin - 1
            pltpu.make_async_copy(y_hbm.at[sidx.at[w]], bufa, sema).wait()
            pltpu.sync_copy(bufa, agg_sh.at[didx.at[w]], add=True)

        plsc.subcore_barrier()

        @pl.when(s < NS - 1)
        def _():
            pltpu.sync_copy(agg_sh.at[pl.ds(s * chunk, chunk)],
                            out_hbm.at[c, pl.ds(s * chunk, chunk)])

        @pl.when(s == NS - 1)
        def _():
            pltpu.sync_copy(agg_sh.at[pl.ds((NS - 1) * chunk, last)],
                            out_hbm.at[c, pl.ds((NS - 1) * chunk, last)])

    return k(y, s2d, d2c, zeros2)


def _dot(a, b):
    return jax.lax.dot_general(a, b, (((1,), (0,)), ((), ())),
                               preferred_element_type=jnp.float32)


def _tc_input_layer(x, in_w, in_b, w1, degT, r):
    n, fi = x.shape
    h = in_w.shape[1]

    def body(x_ref, iw_ref, ib_ref, w1_ref, deg_ref, h0_ref, xw_ref, y_ref):
        h0 = jnp.maximum(_dot(x_ref[...], iw_ref[...]) + ib_ref[...], 0.0)
        h0_ref[...] = h0
        xw = _dot(h0, w1_ref[...])
        xw_ref[...] = xw
        deg = deg_ref[...]
        dinv = lax.rsqrt(deg[:, 0:1] + deg[:, 1:2] + 1.0)
        y_ref[...] = xw * dinv

    grid = (n // r,)
    return pl.pallas_call(
        body,
        grid=grid,
        in_specs=[
            pl.BlockSpec((r, fi), lambda i: (i, 0)),
            pl.BlockSpec((fi, h), lambda i: (0, 0)),
            pl.BlockSpec((1, h), lambda i: (0, 0)),
            pl.BlockSpec((h, h), lambda i: (0, 0)),
            pl.BlockSpec((r, 2), lambda i: (i, 0)),
        ],
        out_specs=[
            pl.BlockSpec((r, h), lambda i: (i, 0)),
            pl.BlockSpec((r, h), lambda i: (i, 0)),
            pl.BlockSpec((r, h), lambda i: (i, 0)),
        ],
        out_shape=[
            jax.ShapeDtypeStruct((n, h), jnp.float32),
            jax.ShapeDtypeStruct((n, h), jnp.float32),
            jax.ShapeDtypeStruct((n, h), jnp.float32),
        ],
    )(x, in_w, in_b, w1, degT)


def _tc_layer_epilogue(aggp, xw, hprev, degT, conv_b, ln_s, ln_b, w_next, r):
    _, half, h = aggp.shape
    n = 2 * half
    have_next = w_next is not None
    nb = half // r

    def body(*refs):
        if have_next:
            (agg_ref, xw_ref, hp_ref, deg_ref, cb_ref, ls_ref, lb_ref,
             wn_ref, h_ref, xw2_ref, y_ref) = refs
        else:
            (agg_ref, xw_ref, hp_ref, deg_ref, cb_ref, ls_ref, lb_ref,
             h_ref, pool_ref) = refs
        deg = deg_ref[...]
        dinv = lax.rsqrt(deg[:, 0:1] + deg[:, 1:2] + 1.0)
        xwv = xw_ref[...]
        agg = agg_ref[0] * dinv + xwv * (dinv * dinv) + cb_ref[...]
        m = jnp.mean(agg, axis=1, keepdims=True)
        cen = agg - m
        var = jnp.mean(cen * cen, axis=1, keepdims=True)
        ln = cen * lax.rsqrt(var + 1e-5) * ls_ref[...] + lb_ref[...]
        hv = jnp.maximum(ln, 0.0) + hp_ref[...]
        h_ref[...] = hv
        if have_next:
            xw2 = _dot(hv, wn_ref[...])
            xw2_ref[...] = xw2
            y_ref[...] = xw2 * dinv
        else:
            i = pl.program_id(0)

            @pl.when(i == 0)
            def _():
                pool_ref[...] = jnp.zeros_like(pool_ref)

            pool_ref[...] += jnp.sum(hv, axis=0, keepdims=True)

    in_specs = [
        pl.BlockSpec((1, r, h), lambda i: (i // nb, i % nb, 0)),
        pl.BlockSpec((r, h), lambda i: (i, 0)),
        pl.BlockSpec((r, h), lambda i: (i, 0)),
        pl.BlockSpec((r, 2), lambda i: (i, 0)),
        pl.BlockSpec((1, h), lambda i: (0, 0)),
        pl.BlockSpec((1, h), lambda i: (0, 0)),
        pl.BlockSpec((1, h), lambda i: (0, 0)),
    ]
    out_specs = [pl.BlockSpec((r, h), lambda i: (i, 0))]
    out_shape = [jax.ShapeDtypeStruct((n, h), jnp.float32)]
    args = [aggp, xw, hprev, degT, conv_b, ln_s, ln_b]
    if have_next:
        in_specs.append(pl.BlockSpec((h, h), lambda i: (0, 0)))
        out_specs += [pl.BlockSpec((r, h), lambda i: (i, 0)),
                      pl.BlockSpec((r, h), lambda i: (i, 0))]
        out_shape += [jax.ShapeDtypeStruct((n, h), jnp.float32),
                      jax.ShapeDtypeStruct((n, h), jnp.float32)]
        args.append(w_next)
    else:
        out_specs.append(pl.BlockSpec((1, h), lambda i: (0, 0)))
        out_shape.append(jax.ShapeDtypeStruct((1, h), jnp.float32))

    return pl.pallas_call(
        body, grid=(n // r,), in_specs=in_specs, out_specs=out_specs,
        out_shape=out_shape,
    )(*args)


def _tc_globals(pooled, gf, glob_w, glob_b, vln_s, vln_b,
                v1_w, v1_b, v2_w, v2_b, v3_w, v3_b,
                s1b, b1, t1, t1b, t2blk, t2b, n_nodes):
    def body(pool_ref, gf_ref, gw_ref, gb_ref, vls_ref, vlb_ref,
             v1w_ref, v1b_ref, v2w_ref, v2b_ref, v3w_ref, v3b_ref,
             s1b_ref, b1_ref, t1_ref, t1b_ref, t2_ref, t2b_ref,
             val_ref, tl_ref, be_ref):
        gemb = jnp.maximum(_dot(gf_ref[...], gw_ref[...]) + gb_ref[...], 0.0)
        graph_emb = pool_ref[...] * (1.0 / n_nodes)
        vin = jnp.concatenate([graph_emb, gemb], axis=1)
        m = jnp.mean(vin, axis=1, keepdims=True)
        cen = vin - m
        var = jnp.mean(cen * cen, axis=1, keepdims=True)
        vin = cen * lax.rsqrt(var + 1e-5) * vls_ref[...] + vlb_ref[...]
        v = jnp.maximum(_dot(vin, v1w_ref[...]) + v1b_ref[...], 0.0)
        v = jnp.maximum(_dot(v, v2w_ref[...]) + v2b_ref[...], 0.0)
        val_ref[...] = _dot(v, v3w_ref[...]) + v3b_ref[...]
        th = jnp.maximum(_dot(gemb, t1_ref[...]) + t1b_ref[...], 0.0)
        tl_ref[...] = _dot(th, t2_ref[...]) + t2b_ref[...]
        be_ref[...] = _dot(gemb, s1b_ref[...]) + b1_ref[...]

    args = [pooled, gf, glob_w, glob_b, vln_s, vln_b,
            v1_w, v1_b, v2_w, v2_b, v3_w, v3_b,
            s1b, b1, t1, t1b, t2blk, t2b]
    return pl.pallas_call(
        body,
        in_specs=[pl.BlockSpec(a.shape, lambda: tuple(0 for _ in a.shape))
                  for a in args],
        out_specs=[
            pl.BlockSpec((1, 1), lambda: (0, 0)),
            pl.BlockSpec(t2b.shape, lambda: (0, 0)),
            pl.BlockSpec(b1.shape, lambda: (0, 0)),
        ],
        out_shape=[
            jax.ShapeDtypeStruct((1, 1), jnp.float32),
            jax.ShapeDtypeStruct(t2b.shape, jnp.float32),
            jax.ShapeDtypeStruct(b1.shape, jnp.float32),
        ],
    )(*args)


def _tc_heads(h3, s1a, bias_eff, w2flat, b2vec, r):
    n, h = h3.shape
    cols = s1a.shape[1]
    nh = cols // h

    def body(h_ref, w_ref, be_ref, w2_ref, b2_ref, out_ref):
        h1 = jnp.maximum(_dot(h_ref[...], w_ref[...]) + be_ref[...], 0.0)
        h1 = h1 * w2_ref[...]
        pieces = [jnp.sum(h1[:, k * h:(k + 1) * h], axis=1, keepdims=True)
                  for k in range(nh)]
        out_ref[...] = jnp.concatenate(pieces, axis=1) + b2_ref[...]

    return pl.pallas_call(
        body,
        grid=(n // r,),
        in_specs=[
            pl.BlockSpec((r, h), lambda i: (i, 0)),
            pl.BlockSpec((h, cols), lambda i: (0, 0)),
            pl.BlockSpec((1, cols), lambda i: (0, 0)),
            pl.BlockSpec((1, cols), lambda i: (0, 0)),
            pl.BlockSpec((1, nh), lambda i: (0, 0)),
        ],
        out_specs=pl.BlockSpec((r, nh), lambda i: (i, 0)),
        out_shape=jax.ShapeDtypeStruct((n, nh), jnp.float32),
    )(h3, s1a, bias_eff, w2flat, b2vec)


def kernel(x, edge_index, global_features, batch, params):
    n, fi = x.shape
    e = edge_index.shape[1]
    h = params['in_w'].shape[1]
    nwin_d = e // (NC * NS * WD)
    nwin_a = e // (NS * WA)
    r = 1000

    dst_d = edge_index[1].reshape(e // WD, WD)
    src_a = edge_index[0].reshape(e // WA, WA)
    half = n // 2
    dst = edge_index[1]
    tile_of = (jnp.arange(e, dtype=jnp.int32) // (e // NS))
    dlo = jnp.where(dst < half, dst, half + tile_of)
    dhi = jnp.where(dst >= half, dst - half, half + tile_of)
    d2c = jnp.stack([dlo, dhi]).reshape(2, e // WA, WA)
    npad = ((n + 2047) // 2048) * 2048
    zeros1 = jnp.zeros((npad,), jnp.float32)
    zeros2 = jnp.zeros((half + NS, h), jnp.float32)

    row = lambda v: v.reshape(1, -1)

    degp = _sc_degree(dst_d, zeros1, npad, nwin_d)
    degT = degp[:, :n].T
    h0, xw, y = _tc_input_layer(x, params['in_w'], row(params['in_b']),
                                params['conv_w'][0], degT, r)

    hcur = h0
    pooled = None
    for i in range(3):
        aggp = _sc_aggregate(y, src_a, d2c, zeros2, n, h, nwin_a)
        w_next = params['conv_w'][i + 1] if i < 2 else None
        outs = _tc_layer_epilogue(
            aggp, xw, hcur, degT, row(params['conv_b'][i]),
            row(params['ln_s'][i]), row(params['ln_b'][i]), w_next, r)
        if i < 2:
            hcur, xw, y = outs
        else:
            hcur, pooled = outs

    heads = params['heads']
    s1a = jnp.concatenate(
        [w for hp in heads for w in (hp['s1_w'][:h], hp['d1_w'][:h])], axis=1)
    s1b = jnp.concatenate(
        [w for hp in heads for w in (hp['s1_w'][h:], hp['d1_w'][h:])], axis=1)
    b1 = jnp.concatenate(
        [b for hp in heads for b in (hp['s1_b'], hp['d1_b'])]).reshape(1, -1)
    w2flat = jnp.concatenate(
        [w[:, 0] for hp in heads for w in (hp['s2_w'], hp['d2_w'])]
    ).reshape(1, -1)
    b2vec = jnp.stack(
        [b[0] for hp in heads for b in (hp['s2_b'], hp['d2_b'])]).reshape(1, -1)
    t1 = jnp.concatenate([hp['t1_w'] for hp in heads], axis=1)
    t1b = jnp.concatenate([hp['t1_b'] for hp in heads]).reshape(1, -1)
    nt = heads[0]['t2_w'].shape[1]
    t2blk = jnp.zeros((len(heads) * h, len(heads) * nt), jnp.float32)
    for k, hp in enumerate(heads):
        t2blk = t2blk.at[k * h:(k + 1) * h, k * nt:(k + 1) * nt].set(hp['t2_w'])
    t2b = jnp.concatenate([hp['t2_b'] for hp in heads]).reshape(1, -1)

    value, tl_all, bias_eff = _tc_globals(
        pooled, global_features, params['glob_w'], row(params['glob_b']),
        row(params['vln_s']), row(params['vln_b']),
        params['v1_w'], row(params['v1_b']), params['v2_w'],
        row(params['v2_b']), params['v3_w'], row(params['v3_b']),
        s1b, b1, t1, t1b, t2blk, t2b, float(n))

    out10 = _tc_heads(hcur, s1a, bias_eff, w2flat, b2vec, r)

    parts = []
    for k in range(len(heads)):
        parts.append(out10[:, 2 * k])
        parts.append(out10[:, 2 * k + 1])
        parts.append(tl_all[0, k * nt:(k + 1) * nt])
    parts.append(value[0])
    return jnp.concatenate(parts)


# speedup vs baseline: 17.2331x; 1.1360x over previous
import functools

import jax
import jax.numpy as jnp
from jax import lax
from jax.experimental import pallas as pl
from jax.experimental.pallas import tpu as pltpu
from jax.experimental.pallas import tpu_sc as plsc

NC = 2
NS = 16
WD = 50
WA = 125
KB = 3


def _sc_degree(d2d, zeros1, npad, nwin):
    mesh = plsc.VectorSubcoreMesh(core_axis_name="c", subcore_axis_name="s")
    chunk = npad // NS

    @functools.partial(
        pl.kernel,
        out_type=jax.ShapeDtypeStruct((NC, npad), jnp.float32),
        mesh=mesh,
        scratch_types=[
            pltpu.VMEM((nwin, WD), jnp.int32),
            pltpu.VMEM((WD,), jnp.float32),
            pltpu.VMEM_SHARED((npad,), jnp.float32),
        ],
    )
    def k(d_hbm, z_hbm, out_hbm, didx, ones, deg_sh):
        c = lax.axis_index("c")
        s = lax.axis_index("s")
        wid = c * NS + s

        pltpu.sync_copy(d_hbm.at[pl.ds(wid * nwin, nwin)], didx)
        for i in range(WD // 16):
            ones[pl.ds(i * 16, 16)] = jnp.ones((16,), jnp.float32)
        ones[pl.ds(WD - 16, 16)] = jnp.ones((16,), jnp.float32)
        pltpu.sync_copy(z_hbm.at[pl.ds(s * chunk, chunk)],
                        deg_sh.at[pl.ds(s * chunk, chunk)])
        plsc.subcore_barrier()

        def body(w, carry):
            pltpu.sync_copy(ones, deg_sh.at[didx.at[w]], add=True)
            return carry

        lax.fori_loop(0, nwin, body, 0)
        plsc.subcore_barrier()
        pltpu.sync_copy(deg_sh.at[pl.ds(s * chunk, chunk)],
                        out_hbm.at[c, pl.ds(s * chunk, chunk)])

    return k(d2d, zeros1)


def _sc_aggregate(y, s2d, d2c, zeros2, n, f, nwin):
    mesh = plsc.VectorSubcoreMesh(core_axis_name="c", subcore_axis_name="s")
    half = n // 2
    arows = half + NS
    chunk = (half // NS) // 8 * 8
    last = half - chunk * (NS - 1)
    zchunk = (arows // NS) // 8 * 8
    zlast = arows - zchunk * (NS - 1)

    @functools.partial(
        pl.kernel,
        out_type=jax.ShapeDtypeStruct((NC, half, f), jnp.float32),
        mesh=mesh,
        scratch_types=[
            pltpu.VMEM((nwin, WA), jnp.int32),
            pltpu.VMEM((nwin, WA), jnp.int32),
            pltpu.VMEM((WA, f), jnp.float32),
            pltpu.VMEM((WA, f), jnp.float32),
            pltpu.VMEM((WA, f), jnp.float32),
            pltpu.VMEM_SHARED((arows, f), jnp.float32),
            pltpu.SemaphoreType.DMA,
            pltpu.SemaphoreType.DMA,
            pltpu.SemaphoreType.DMA,
        ],
    )
    def k(y_hbm, s_hbm, d_hbm, z_hbm, out_hbm,
          sidx, didx, b0, b1, b2, agg_sh, s0, s1, s2):
        bufs = [b0, b1, b2]
        sems = [s0, s1, s2]
        c = lax.axis_index("c")
        s = lax.axis_index("s")

        pltpu.sync_copy(s_hbm.at[pl.ds(s * nwin, nwin)], sidx)
        pltpu.sync_copy(d_hbm.at[c, pl.ds(s * nwin, nwin)], didx)

        @pl.when(s < NS - 1)
        def _():
            pltpu.sync_copy(z_hbm.at[pl.ds(s * zchunk, zchunk)],
                            agg_sh.at[pl.ds(s * zchunk, zchunk)])

        @pl.when(s == NS - 1)
        def _():
            pltpu.sync_copy(z_hbm.at[pl.ds((NS - 1) * zchunk, zlast)],
                            agg_sh.at[pl.ds((NS - 1) * zchunk, zlast)])

        plsc.subcore_barrier()

        for b in range(KB):
            pltpu.async_copy(y_hbm.at[sidx.at[b]], bufs[b], sems[b])

        def body(g, carry):
            for b in range(KB):
                w = KB * g + b
                pltpu.make_async_copy(
                    y_hbm.at[sidx.at[w]], bufs[b], sems[b]).wait()
                pltpu.sync_copy(bufs[b], agg_sh.at[didx.at[w]], add=True)

                @pl.when(w + KB < nwin)
                def _():
                    pltpu.async_copy(
                        y_hbm.at[sidx.at[w + KB]], bufs[b], sems[b])

            return carry

        lax.fori_loop(0, nwin // KB, body, 0)
        for b in range(nwin % KB):
            w = (nwin // KB) * KB + b
            pltpu.make_async_copy(y_hbm.at[sidx.at[w]], bufs[b], sems[b]).wait()
            pltpu.sync_copy(bufs[b], agg_sh.at[didx.at[w]], add=True)

        plsc.subcore_barrier()

        @pl.when(s < NS - 1)
        def _():
            pltpu.sync_copy(agg_sh.at[pl.ds(s * chunk, chunk)],
                            out_hbm.at[c, pl.ds(s * chunk, chunk)])

        @pl.when(s == NS - 1)
        def _():
            pltpu.sync_copy(agg_sh.at[pl.ds((NS - 1) * chunk, last)],
                            out_hbm.at[c, pl.ds((NS - 1) * chunk, last)])

    return k(y, s2d, d2c, zeros2)


def _dot(a, b):
    return jax.lax.dot_general(a, b, (((1,), (0,)), ((), ())),
                               preferred_element_type=jnp.float32)


def _tc_input_layer(x, in_w, in_b, w1, degT, r):
    n, fi = x.shape
    h = in_w.shape[1]

    def body(x_ref, iw_ref, ib_ref, w1_ref, deg_ref, h0_ref, xw_ref, y_ref):
        h0 = jnp.maximum(_dot(x_ref[...], iw_ref[...]) + ib_ref[...], 0.0)
        h0_ref[...] = h0
        xw = _dot(h0, w1_ref[...])
        xw_ref[...] = xw
        deg = deg_ref[...]
        dinv = lax.rsqrt(deg[:, 0:1] + deg[:, 1:2] + 1.0)
        y_ref[...] = xw * dinv

    grid = (n // r,)
    return pl.pallas_call(
        body,
        grid=grid,
        in_specs=[
            pl.BlockSpec((r, fi), lambda i: (i, 0)),
            pl.BlockSpec((fi, h), lambda i: (0, 0)),
            pl.BlockSpec((1, h), lambda i: (0, 0)),
            pl.BlockSpec((h, h), lambda i: (0, 0)),
            pl.BlockSpec((r, 2), lambda i: (i, 0)),
        ],
        out_specs=[
            pl.BlockSpec((r, h), lambda i: (i, 0)),
            pl.BlockSpec((r, h), lambda i: (i, 0)),
            pl.BlockSpec((r, h), lambda i: (i, 0)),
        ],
        out_shape=[
            jax.ShapeDtypeStruct((n, h), jnp.float32),
            jax.ShapeDtypeStruct((n, h), jnp.float32),
            jax.ShapeDtypeStruct((n, h), jnp.float32),
        ],
    )(x, in_w, in_b, w1, degT)


def _tc_layer_epilogue(aggp, xw, hprev, degT, conv_b, ln_s, ln_b, w_next, r):
    _, half, h = aggp.shape
    n = 2 * half
    have_next = w_next is not None
    nb = half // r

    def body(*refs):
        if have_next:
            (agg_ref, xw_ref, hp_ref, deg_ref, cb_ref, ls_ref, lb_ref,
             wn_ref, h_ref, xw2_ref, y_ref) = refs
        else:
            (agg_ref, xw_ref, hp_ref, deg_ref, cb_ref, ls_ref, lb_ref,
             h_ref, pool_ref) = refs
        deg = deg_ref[...]
        dinv = lax.rsqrt(deg[:, 0:1] + deg[:, 1:2] + 1.0)
        xwv = xw_ref[...]
        agg = agg_ref[0] * dinv + xwv * (dinv * dinv) + cb_ref[...]
        m = jnp.mean(agg, axis=1, keepdims=True)
        cen = agg - m
        var = jnp.mean(cen * cen, axis=1, keepdims=True)
        ln = cen * lax.rsqrt(var + 1e-5) * ls_ref[...] + lb_ref[...]
        hv = jnp.maximum(ln, 0.0) + hp_ref[...]
        h_ref[...] = hv
        if have_next:
            xw2 = _dot(hv, wn_ref[...])
            xw2_ref[...] = xw2
            y_ref[...] = xw2 * dinv
        else:
            i = pl.program_id(0)

            @pl.when(i == 0)
            def _():
                pool_ref[...] = jnp.zeros_like(pool_ref)

            pool_ref[...] += jnp.sum(hv, axis=0, keepdims=True)

    in_specs = [
        pl.BlockSpec((1, r, h), lambda i: (i // nb, i % nb, 0)),
        pl.BlockSpec((r, h), lambda i: (i, 0)),
        pl.BlockSpec((r, h), lambda i: (i, 0)),
        pl.BlockSpec((r, 2), lambda i: (i, 0)),
        pl.BlockSpec((1, h), lambda i: (0, 0)),
        pl.BlockSpec((1, h), lambda i: (0, 0)),
        pl.BlockSpec((1, h), lambda i: (0, 0)),
    ]
    out_specs = [pl.BlockSpec((r, h), lambda i: (i, 0))]
    out_shape = [jax.ShapeDtypeStruct((n, h), jnp.float32)]
    args = [aggp, xw, hprev, degT, conv_b, ln_s, ln_b]
    if have_next:
        in_specs.append(pl.BlockSpec((h, h), lambda i: (0, 0)))
        out_specs += [pl.BlockSpec((r, h), lambda i: (i, 0)),
                      pl.BlockSpec((r, h), lambda i: (i, 0))]
        out_shape += [jax.ShapeDtypeStruct((n, h), jnp.float32),
                      jax.ShapeDtypeStruct((n, h), jnp.float32)]
        args.append(w_next)
    else:
        out_specs.append(pl.BlockSpec((1, h), lambda i: (0, 0)))
        out_shape.append(jax.ShapeDtypeStruct((1, h), jnp.float32))

    return pl.pallas_call(
        body, grid=(n // r,), in_specs=in_specs, out_specs=out_specs,
        out_shape=out_shape,
    )(*args)


def _tc_globals(pooled, gf, glob_w, glob_b, vln_s, vln_b,
                v1_w, v1_b, v2_w, v2_b, v3_w, v3_b,
                s1b, b1, t1, t1b, t2blk, t2b, n_nodes):
    def body(pool_ref, gf_ref, gw_ref, gb_ref, vls_ref, vlb_ref,
             v1w_ref, v1b_ref, v2w_ref, v2b_ref, v3w_ref, v3b_ref,
             s1b_ref, b1_ref, t1_ref, t1b_ref, t2_ref, t2b_ref,
             val_ref, tl_ref, be_ref):
        gemb = jnp.maximum(_dot(gf_ref[...], gw_ref[...]) + gb_ref[...], 0.0)
        graph_emb = pool_ref[...] * (1.0 / n_nodes)
        vin = jnp.concatenate([graph_emb, gemb], axis=1)
        m = jnp.mean(vin, axis=1, keepdims=True)
        cen = vin - m
        var = jnp.mean(cen * cen, axis=1, keepdims=True)
        vin = cen * lax.rsqrt(var + 1e-5) * vls_ref[...] + vlb_ref[...]
        v = jnp.maximum(_dot(vin, v1w_ref[...]) + v1b_ref[...], 0.0)
        v = jnp.maximum(_dot(v, v2w_ref[...]) + v2b_ref[...], 0.0)
        val_ref[...] = _dot(v, v3w_ref[...]) + v3b_ref[...]
        th = jnp.maximum(_dot(gemb, t1_ref[...]) + t1b_ref[...], 0.0)
        tl_ref[...] = _dot(th, t2_ref[...]) + t2b_ref[...]
        be_ref[...] = _dot(gemb, s1b_ref[...]) + b1_ref[...]

    args = [pooled, gf, glob_w, glob_b, vln_s, vln_b,
            v1_w, v1_b, v2_w, v2_b, v3_w, v3_b,
            s1b, b1, t1, t1b, t2blk, t2b]
    return pl.pallas_call(
        body,
        in_specs=[pl.BlockSpec(a.shape, lambda: tuple(0 for _ in a.shape))
                  for a in args],
        out_specs=[
            pl.BlockSpec((1, 1), lambda: (0, 0)),
            pl.BlockSpec(t2b.shape, lambda: (0, 0)),
            pl.BlockSpec(b1.shape, lambda: (0, 0)),
        ],
        out_shape=[
            jax.ShapeDtypeStruct((1, 1), jnp.float32),
            jax.ShapeDtypeStruct(t2b.shape, jnp.float32),
            jax.ShapeDtypeStruct(b1.shape, jnp.float32),
        ],
    )(*args)


def _tc_heads(h3, s1a, bias_eff, w2flat, b2vec, r):
    n, h = h3.shape
    cols = s1a.shape[1]
    nh = cols // h

    def body(h_ref, w_ref, be_ref, w2_ref, b2_ref, out_ref):
        h1 = jnp.maximum(_dot(h_ref[...], w_ref[...]) + be_ref[...], 0.0)
        h1 = h1 * w2_ref[...]
        pieces = [jnp.sum(h1[:, k * h:(k + 1) * h], axis=1, keepdims=True)
                  for k in range(nh)]
        out_ref[...] = jnp.concatenate(pieces, axis=1) + b2_ref[...]

    return pl.pallas_call(
        body,
        grid=(n // r,),
        in_specs=[
            pl.BlockSpec((r, h), lambda i: (i, 0)),
            pl.BlockSpec((h, cols), lambda i: (0, 0)),
            pl.BlockSpec((1, cols), lambda i: (0, 0)),
            pl.BlockSpec((1, cols), lambda i: (0, 0)),
            pl.BlockSpec((1, nh), lambda i: (0, 0)),
        ],
        out_specs=pl.BlockSpec((r, nh), lambda i: (i, 0)),
        out_shape=jax.ShapeDtypeStruct((n, nh), jnp.float32),
    )(h3, s1a, bias_eff, w2flat, b2vec)


def kernel(x, edge_index, global_features, batch, params):
    n, fi = x.shape
    e = edge_index.shape[1]
    h = params['in_w'].shape[1]
    nwin_d = e // (NC * NS * WD)
    nwin_a = e // (NS * WA)
    r = 1000

    dst_d = edge_index[1].reshape(e // WD, WD)
    src_a = edge_index[0].reshape(e // WA, WA)
    half = n // 2
    dst = edge_index[1]
    tile_of = (jnp.arange(e, dtype=jnp.int32) // (e // NS))
    dlo = jnp.where(dst < half, dst, half + tile_of)
    dhi = jnp.where(dst >= half, dst - half, half + tile_of)
    d2c = jnp.stack([dlo, dhi]).reshape(2, e // WA, WA)
    npad = ((n + 2047) // 2048) * 2048
    zeros1 = jnp.zeros((npad,), jnp.float32)
    zeros2 = jnp.zeros((half + NS, h), jnp.float32)

    row = lambda v: v.reshape(1, -1)

    degp = _sc_degree(dst_d, zeros1, npad, nwin_d)
    degT = degp[:, :n].T
    h0, xw, y = _tc_input_layer(x, params['in_w'], row(params['in_b']),
                                params['conv_w'][0], degT, r)

    hcur = h0
    pooled = None
    for i in range(3):
        aggp = _sc_aggregate(y, src_a, d2c, zeros2, n, h, nwin_a)
        w_next = params['conv_w'][i + 1] if i < 2 else None
        outs = _tc_layer_epilogue(
            aggp, xw, hcur, degT, row(params['conv_b'][i]),
            row(params['ln_s'][i]), row(params['ln_b'][i]), w_next, r)
        if i < 2:
            hcur, xw, y = outs
        else:
            hcur, pooled = outs

    heads = params['heads']
    s1a = jnp.concatenate(
        [w for hp in heads for w in (hp['s1_w'][:h], hp['d1_w'][:h])], axis=1)
    s1b = jnp.concatenate(
        [w for hp in heads for w in (hp['s1_w'][h:], hp['d1_w'][h:])], axis=1)
    b1 = jnp.concatenate(
        [b for hp in heads for b in (hp['s1_b'], hp['d1_b'])]).reshape(1, -1)
    w2flat = jnp.concatenate(
        [w[:, 0] for hp in heads for w in (hp['s2_w'], hp['d2_w'])]
    ).reshape(1, -1)
    b2vec = jnp.stack(
        [b[0] for hp in heads for b in (hp['s2_b'], hp['d2_b'])]).reshape(1, -1)
    t1 = jnp.concatenate([hp['t1_w'] for hp in heads], axis=1)
    t1b = jnp.concatenate([hp['t1_b'] for hp in heads]).reshape(1, -1)
    nt = heads[0]['t2_w'].shape[1]
    t2blk = jnp.zeros((len(heads) * h, len(heads) * nt), jnp.float32)
    for k, hp in enumerate(heads):
        t2blk = t2blk.at[k * h:(k + 1) * h, k * nt:(k + 1) * nt].set(hp['t2_w'])
    t2b = jnp.concatenate([hp['t2_b'] for hp in heads]).reshape(1, -1)

    value, tl_all, bias_eff = _tc_globals(
        pooled, global_features, params['glob_w'], row(params['glob_b']),
        row(params['vln_s']), row(params['vln_b']),
        params['v1_w'], row(params['v1_b']), params['v2_w'],
        row(params['v2_b']), params['v3_w'], row(params['v3_b']),
        s1b, b1, t1, t1b, t2blk, t2b, float(n))

    out10 = _tc_heads(hcur, s1a, bias_eff, w2flat, b2vec, r)

    parts = []
    for k in range(len(heads)):
        parts.append(out10[:, 2 * k])
        parts.append(out10[:, 2 * k + 1])
        parts.append(tl_all[0, k * nt:(k + 1) * nt])
    parts.append(value[0])
    return jnp.concatenate(parts)


# speedup vs baseline: 17.5646x; 1.0192x over previous
import functools

import jax
import jax.numpy as jnp
from jax import lax
from jax.experimental import pallas as pl
from jax.experimental.pallas import tpu as pltpu
from jax.experimental.pallas import tpu_sc as plsc

NC = 2
NS = 16
WD = 50
WA = 125
KB = 3
TR = 3


def _sc_degree(d2d, zeros1, npad, nwin):
    mesh = plsc.VectorSubcoreMesh(core_axis_name="c", subcore_axis_name="s")
    chunk = npad // NS

    @functools.partial(
        pl.kernel,
        out_type=jax.ShapeDtypeStruct((NC, npad), jnp.float32),
        mesh=mesh,
        scratch_types=[
            pltpu.VMEM((nwin, WD), jnp.int32),
            pltpu.VMEM((WD,), jnp.float32),
            pltpu.VMEM_SHARED((npad,), jnp.float32),
        ],
    )
    def k(d_hbm, z_hbm, out_hbm, didx, ones, deg_sh):
        c = lax.axis_index("c")
        s = lax.axis_index("s")
        wid = c * NS + s

        pltpu.sync_copy(d_hbm.at[pl.ds(wid * nwin, nwin)], didx)
        for i in range(WD // 16):
            ones[pl.ds(i * 16, 16)] = jnp.ones((16,), jnp.float32)
        ones[pl.ds(WD - 16, 16)] = jnp.ones((16,), jnp.float32)
        pltpu.sync_copy(z_hbm.at[pl.ds(s * chunk, chunk)],
                        deg_sh.at[pl.ds(s * chunk, chunk)])
        plsc.subcore_barrier()

        def body(w, carry):
            pltpu.sync_copy(ones, deg_sh.at[didx.at[w]], add=True)
            return carry

        lax.fori_loop(0, nwin, body, 0)
        plsc.subcore_barrier()
        pltpu.sync_copy(deg_sh.at[pl.ds(s * chunk, chunk)],
                        out_hbm.at[c, pl.ds(s * chunk, chunk)])

    return k(d2d, zeros1)


def _sc_aggregate(y, s2d, d2c, zeros2, n, f, nwin):
    mesh = plsc.VectorSubcoreMesh(core_axis_name="c", subcore_axis_name="s")
    half = n // 2
    arows = half + NS * TR
    chunk = (half // NS) // 8 * 8
    last = half - chunk * (NS - 1)
    zchunk = (arows // NS) // 8 * 8
    zlast = arows - zchunk * (NS - 1)

    @functools.partial(
        pl.kernel,
        out_type=jax.ShapeDtypeStruct((NC, half, f), jnp.float32),
        mesh=mesh,
        scratch_types=[
            pltpu.VMEM((nwin, WA), jnp.int32),
            pltpu.VMEM((nwin, WA), jnp.int32),
            pltpu.VMEM((WA, f), jnp.float32),
            pltpu.VMEM((WA, f), jnp.float32),
            pltpu.VMEM((WA, f), jnp.float32),
            pltpu.VMEM_SHARED((arows, f), jnp.float32),
            pltpu.SemaphoreType.DMA,
            pltpu.SemaphoreType.DMA,
            pltpu.SemaphoreType.DMA,
        ],
    )
    def k(y_hbm, s_hbm, d_hbm, z_hbm, out_hbm,
          sidx, didx, b0, b1, b2, agg_sh, s0, s1, s2):
        bufs = [b0, b1, b2]
        sems = [s0, s1, s2]
        c = lax.axis_index("c")
        s = lax.axis_index("s")

        pltpu.sync_copy(s_hbm.at[pl.ds(s * nwin, nwin)], sidx)
        pltpu.sync_copy(d_hbm.at[c, pl.ds(s * nwin, nwin)], didx)

        @pl.when(s < NS - 1)
        def _():
            pltpu.sync_copy(z_hbm.at[pl.ds(s * zchunk, zchunk)],
                            agg_sh.at[pl.ds(s * zchunk, zchunk)])

        @pl.when(s == NS - 1)
        def _():
            pltpu.sync_copy(z_hbm.at[pl.ds((NS - 1) * zchunk, zlast)],
                            agg_sh.at[pl.ds((NS - 1) * zchunk, zlast)])

        plsc.subcore_barrier()

        for b in range(KB):
            pltpu.async_copy(y_hbm.at[sidx.at[b]], bufs[b], sems[b])

        def body(g, carry):
            for b in range(KB):
                w = KB * g + b
                pltpu.make_async_copy(
                    y_hbm.at[sidx.at[w]], bufs[b], sems[b]).wait()
                pltpu.sync_copy(bufs[b], agg_sh.at[didx.at[w]], add=True)

                @pl.when(w + KB < nwin)
                def _():
                    pltpu.async_copy(
                        y_hbm.at[sidx.at[w + KB]], bufs[b], sems[b])

            return carry

        lax.fori_loop(0, nwin // KB, body, 0)
        for b in range(nwin % KB):
            w = (nwin // KB) * KB + b
            pltpu.make_async_copy(y_hbm.at[sidx.at[w]], bufs[b], sems[b]).wait()
            pltpu.sync_copy(bufs[b], agg_sh.at[didx.at[w]], add=True)

        plsc.subcore_barrier()

        @pl.when(s < NS - 1)
        def _():
            pltpu.sync_copy(agg_sh.at[pl.ds(s * chunk, chunk)],
                            out_hbm.at[c, pl.ds(s * chunk, chunk)])

        @pl.when(s == NS - 1)
        def _():
            pltpu.sync_copy(agg_sh.at[pl.ds((NS - 1) * chunk, last)],
                            out_hbm.at[c, pl.ds((NS - 1) * chunk, last)])

    return k(y, s2d, d2c, zeros2)


def _dot(a, b):
    return jax.lax.dot_general(a, b, (((1,), (0,)), ((), ())),
                               preferred_element_type=jnp.float32)


def _tc_input_layer(x, in_w, in_b, w1, degT, r):
    n, fi = x.shape
    h = in_w.shape[1]

    def body(x_ref, iw_ref, ib_ref, w1_ref, deg_ref, h0_ref, xw_ref, y_ref):
        h0 = jnp.maximum(_dot(x_ref[...], iw_ref[...]) + ib_ref[...], 0.0)
        h0_ref[...] = h0
        xw = _dot(h0, w1_ref[...])
        xw_ref[...] = xw
        deg = deg_ref[...]
        dinv = lax.rsqrt(deg[:, 0:1] + deg[:, 1:2] + 1.0)
        y_ref[...] = xw * dinv

    grid = (n // r,)
    return pl.pallas_call(
        body,
        grid=grid,
        in_specs=[
            pl.BlockSpec((r, fi), lambda i: (i, 0)),
            pl.BlockSpec((fi, h), lambda i: (0, 0)),
            pl.BlockSpec((1, h), lambda i: (0, 0)),
            pl.BlockSpec((h, h), lambda i: (0, 0)),
            pl.BlockSpec((r, 2), lambda i: (i, 0)),
        ],
        out_specs=[
            pl.BlockSpec((r, h), lambda i: (i, 0)),
            pl.BlockSpec((r, h), lambda i: (i, 0)),
            pl.BlockSpec((r, h), lambda i: (i, 0)),
        ],
        out_shape=[
            jax.ShapeDtypeStruct((n, h), jnp.float32),
            jax.ShapeDtypeStruct((n, h), jnp.float32),
            jax.ShapeDtypeStruct((n, h), jnp.float32),
        ],
    )(x, in_w, in_b, w1, degT)


def _layer_core(agg_ref, xw_ref, hp_ref, deg_ref, cb_ref, ls_ref, lb_ref):
    deg = deg_ref[...]
    dinv = lax.rsqrt(deg[:, 0:1] + deg[:, 1:2] + 1.0)
    agg = agg_ref[0] * dinv + xw_ref[...] * (dinv * dinv) + cb_ref[...]
    m = jnp.mean(agg, axis=1, keepdims=True)
    cen = agg - m
    var = jnp.mean(cen * cen, axis=1, keepdims=True)
    ln = cen * lax.rsqrt(var + 1e-5) * ls_ref[...] + lb_ref[...]
    return jnp.maximum(ln, 0.0) + hp_ref[...], dinv


def _epilogue_specs(half, h, r):
    nb = half // r
    return [
        pl.BlockSpec((1, r, h), lambda i: (i // nb, i % nb, 0)),
        pl.BlockSpec((r, h), lambda i: (i, 0)),
        pl.BlockSpec((r, h), lambda i: (i, 0)),
        pl.BlockSpec((r, 2), lambda i: (i, 0)),
        pl.BlockSpec((1, h), lambda i: (0, 0)),
        pl.BlockSpec((1, h), lambda i: (0, 0)),
        pl.BlockSpec((1, h), lambda i: (0, 0)),
    ]


def _tc_layer_epilogue(aggp, xw, hprev, degT, conv_b, ln_s, ln_b, w_next, r):
    _, half, h = aggp.shape
    n = 2 * half

    def body(agg_ref, xw_ref, hp_ref, deg_ref, cb_ref, ls_ref, lb_ref,
             wn_ref, h_ref, xw2_ref, y_ref):
        hv, dinv = _layer_core(agg_ref, xw_ref, hp_ref, deg_ref, cb_ref,
                               ls_ref, lb_ref)
        h_ref[...] = hv
        xw2 = _dot(hv, wn_ref[...])
        xw2_ref[...] = xw2
        y_ref[...] = xw2 * dinv

    in_specs = _epilogue_specs(half, h, r)
    in_specs.append(pl.BlockSpec((h, h), lambda i: (0, 0)))
    return pl.pallas_call(
        body, grid=(n // r,), in_specs=in_specs,
        out_specs=[pl.BlockSpec((r, h), lambda i: (i, 0))] * 3,
        out_shape=[jax.ShapeDtypeStruct((n, h), jnp.float32)] * 3,
    )(aggp, xw, hprev, degT, conv_b, ln_s, ln_b, w_next)


def _tc_final_layer(aggp, xw, hprev, degT, conv_b, ln_s, ln_b,
                    s1a, bias_eff, w2flat, b2vec, r):
    _, half, h = aggp.shape
    n = 2 * half
    cols = s1a.shape[1]
    nh = cols // h

    def body(agg_ref, xw_ref, hp_ref, deg_ref, cb_ref, ls_ref, lb_ref,
             w_ref, be_ref, w2_ref, b2_ref, out_ref, pool_ref):
        hv, _ = _layer_core(agg_ref, xw_ref, hp_ref, deg_ref, cb_ref,
                            ls_ref, lb_ref)
        i = pl.program_id(0)

        @pl.when(i == 0)
        def _():
            pool_ref[...] = jnp.zeros_like(pool_ref)

        pool_ref[...] += jnp.sum(hv, axis=0, keepdims=True)
        h1 = jnp.maximum(_dot(hv, w_ref[...]) + be_ref[...], 0.0)
        h1 = h1 * w2_ref[...]
        pieces = [jnp.sum(h1[:, k * h:(k + 1) * h], axis=1, keepdims=True)
                  for k in range(nh)]
        out_ref[...] = jnp.concatenate(pieces, axis=1) + b2_ref[...]

    in_specs = _epilogue_specs(half, h, r)
    in_specs += [
        pl.BlockSpec((h, cols), lambda i: (0, 0)),
        pl.BlockSpec((1, cols), lambda i: (0, 0)),
        pl.BlockSpec((1, cols), lambda i: (0, 0)),
        pl.BlockSpec((1, nh), lambda i: (0, 0)),
    ]
    return pl.pallas_call(
        body, grid=(n // r,), in_specs=in_specs,
        out_specs=[pl.BlockSpec((r, nh), lambda i: (i, 0)),
                   pl.BlockSpec((1, h), lambda i: (0, 0))],
        out_shape=[jax.ShapeDtypeStruct((n, nh), jnp.float32),
                   jax.ShapeDtypeStruct((1, h), jnp.float32)],
    )(aggp, xw, hprev, degT, conv_b, ln_s, ln_b,
      s1a, bias_eff, w2flat, b2vec)


def _tc_globals(gf, glob_w, glob_b, s1b, b1, t1, t1b, t2blk, t2b):
    def body(gf_ref, gw_ref, gb_ref, s1b_ref, b1_ref,
             t1_ref, t1b_ref, t2_ref, t2b_ref,
             gemb_ref, tl_ref, be_ref):
        gemb = jnp.maximum(_dot(gf_ref[...], gw_ref[...]) + gb_ref[...], 0.0)
        gemb_ref[...] = gemb
        th = jnp.maximum(_dot(gemb, t1_ref[...]) + t1b_ref[...], 0.0)
        tl_ref[...] = _dot(th, t2_ref[...]) + t2b_ref[...]
        be_ref[...] = _dot(gemb, s1b_ref[...]) + b1_ref[...]

    args = [gf, glob_w, glob_b, s1b, b1, t1, t1b, t2blk, t2b]
    return pl.pallas_call(
        body,
        in_specs=[pl.BlockSpec(a.shape, lambda: tuple(0 for _ in a.shape))
                  for a in args],
        out_specs=[
            pl.BlockSpec(glob_b.shape, lambda: (0, 0)),
            pl.BlockSpec(t2b.shape, lambda: (0, 0)),
            pl.BlockSpec(b1.shape, lambda: (0, 0)),
        ],
        out_shape=[
            jax.ShapeDtypeStruct(glob_b.shape, jnp.float32),
            jax.ShapeDtypeStruct(t2b.shape, jnp.float32),
            jax.ShapeDtypeStruct(b1.shape, jnp.float32),
        ],
    )(*args)


def _tc_value(pooled, gemb, vln_s, vln_b,
              v1_w, v1_b, v2_w, v2_b, v3_w, v3_b, n_nodes):
    def body(pool_ref, gemb_ref, vls_ref, vlb_ref,
             v1w_ref, v1b_ref, v2w_ref, v2b_ref, v3w_ref, v3b_ref,
             val_ref):
        graph_emb = pool_ref[...] * (1.0 / n_nodes)
        vin = jnp.concatenate([graph_emb, gemb_ref[...]], axis=1)
        m = jnp.mean(vin, axis=1, keepdims=True)
        cen = vin - m
        var = jnp.mean(cen * cen, axis=1, keepdims=True)
        vin = cen * lax.rsqrt(var + 1e-5) * vls_ref[...] + vlb_ref[...]
        v = jnp.maximum(_dot(vin, v1w_ref[...]) + v1b_ref[...], 0.0)
        v = jnp.maximum(_dot(v, v2w_ref[...]) + v2b_ref[...], 0.0)
        val_ref[...] = _dot(v, v3w_ref[...]) + v3b_ref[...]

    args = [pooled, gemb, vln_s, vln_b,
            v1_w, v1_b, v2_w, v2_b, v3_w, v3_b]
    return pl.pallas_call(
        body,
        in_specs=[pl.BlockSpec(a.shape, lambda: tuple(0 for _ in a.shape))
                  for a in args],
        out_specs=pl.BlockSpec((1, 1), lambda: (0, 0)),
        out_shape=jax.ShapeDtypeStruct((1, 1), jnp.float32),
    )(*args)


def kernel(x, edge_index, global_features, batch, params):
    n, fi = x.shape
    e = edge_index.shape[1]
    h = params['in_w'].shape[1]
    nwin_d = e // (NC * NS * WD)
    nwin_a = e // (NS * WA)
    r = 1000

    dst_d = edge_index[1].reshape(e // WD, WD)
    src_a = edge_index[0].reshape(e // WA, WA)
    half = n // 2
    dst = edge_index[1]
    pos = jnp.arange(e, dtype=jnp.int32)
    trash = half + (pos // (e // NS)) * TR + pos % TR
    dlo = jnp.where(dst < half, dst, trash)
    dhi = jnp.where(dst >= half, dst - half, trash)
    d2c = jnp.stack([dlo, dhi]).reshape(2, e // WA, WA)
    npad = ((n + 2047) // 2048) * 2048
    zeros1 = jnp.zeros((npad,), jnp.float32)
    zeros2 = jnp.zeros((half + NS * TR, h), jnp.float32)

    row = lambda v: v.reshape(1, -1)
    heads = params['heads']
    s1a = jnp.concatenate(
        [w for hp in heads for w in (hp['s1_w'][:h], hp['d1_w'][:h])], axis=1)
    s1b = jnp.concatenate(
        [w for hp in heads for w in (hp['s1_w'][h:], hp['d1_w'][h:])], axis=1)
    b1 = jnp.concatenate(
        [b for hp in heads for b in (hp['s1_b'], hp['d1_b'])]).reshape(1, -1)
    w2flat = jnp.concatenate(
        [w[:, 0] for hp in heads for w in (hp['s2_w'], hp['d2_w'])]
    ).reshape(1, -1)
    b2vec = jnp.stack(
        [b[0] for hp in heads for b in (hp['s2_b'], hp['d2_b'])]).reshape(1, -1)
    t1 = jnp.concatenate([hp['t1_w'] for hp in heads], axis=1)
    t1b = jnp.concatenate([hp['t1_b'] for hp in heads]).reshape(1, -1)
    nt = heads[0]['t2_w'].shape[1]
    t2blk = jnp.zeros((len(heads) * h, len(heads) * nt), jnp.float32)
    for k, hp in enumerate(heads):
        t2blk = t2blk.at[k * h:(k + 1) * h, k * nt:(k + 1) * nt].set(hp['t2_w'])
    t2b = jnp.concatenate([hp['t2_b'] for hp in heads]).reshape(1, -1)

    gemb, tl_all, bias_eff = _tc_globals(
        global_features, params['glob_w'], row(params['glob_b']),
        s1b, b1, t1, t1b, t2blk, t2b)

    degp = _sc_degree(dst_d, zeros1, npad, nwin_d)
    degT = degp[:, :n].T
    hcur, xw, y = _tc_input_layer(x, params['in_w'], row(params['in_b']),
                                  params['conv_w'][0], degT, r)

    for i in range(2):
        aggp = _sc_aggregate(y, src_a, d2c, zeros2, n, h, nwin_a)
        hcur, xw, y = _tc_layer_epilogue(
            aggp, xw, hcur, degT, row(params['conv_b'][i]),
            row(params['ln_s'][i]), row(params['ln_b'][i]),
            params['conv_w'][i + 1], r)

    aggp = _sc_aggregate(y, src_a, d2c, zeros2, n, h, nwin_a)
    out10, pooled = _tc_final_layer(
        aggp, xw, hcur, degT, row(params['conv_b'][2]),
        row(params['ln_s'][2]), row(params['ln_b'][2]),
        s1a, bias_eff, w2flat, b2vec, r)

    value = _tc_value(
        pooled, gemb, row(params['vln_s']), row(params['vln_b']),
        params['v1_w'], row(params['v1_b']), params['v2_w'],
        row(params['v2_b']), params['v3_w'], row(params['v3_b']), float(n))

    parts = []
    for k in range(len(heads)):
        parts.append(out10[:, 2 * k])
        parts.append(out10[:, 2 * k + 1])
        parts.append(tl_all[0, k * nt:(k + 1) * nt])
    parts.append(value[0])
    return jnp.concatenate(parts)
